# Initial kernel scaffold; baseline (speedup 1.0000x reference)
import jax
import jax.numpy as jnp
from jax.experimental import pallas as pl


def kernel(x, embed_mu, embed_logvar):
    raise NotImplementedError("write your pallas kernel here")


# speedup vs baseline: 2.5828x; 2.5828x over previous
import functools

import jax
import jax.numpy as jnp
from jax import lax
from jax.experimental import pallas as pl
from jax.experimental.pallas import tpu as pltpu
from jax.experimental.pallas import tpu_sc as plsc

Z_N = 16
BATCH = 16384
HIST = 200
B_TOTAL = BATCH * HIST

NUM_CORES = 2
NUM_SUBCORES = 16
NUM_WORKERS = NUM_CORES * NUM_SUBCORES

IDX_W = 128
CHUNK = 1024
SUB = CHUNK // IDX_W
B_PER_W = B_TOTAL // NUM_WORKERS
N_CHUNKS = B_PER_W // CHUNK


def _gather_kernel(x_hbm, mu_hbm, lv_hbm, out_mu, out_lv,
                   idx_buf, rows_mu, rows_lv, sem):
    wid = lax.axis_index("s") * NUM_CORES + lax.axis_index("c")
    base = wid * B_PER_W
    row_base = wid * (B_PER_W // IDX_W)

    def body(i, carry):
        start = base + i * CHUNK
        row = row_base + i * SUB
        pltpu.sync_copy(x_hbm.at[pl.ds(row, SUB)], idx_buf)
        handles = []
        for j in range(SUB):
            handles.append(pltpu.async_copy(
                mu_hbm.at[idx_buf.at[j]],
                rows_mu.at[pl.ds(j * IDX_W, IDX_W)], sem))
            handles.append(pltpu.async_copy(
                lv_hbm.at[idx_buf.at[j]],
                rows_lv.at[pl.ds(j * IDX_W, IDX_W)], sem))
        for h in handles:
            h.wait()
        pltpu.sync_copy(rows_mu, out_mu.at[pl.ds(start, CHUNK)])
        pltpu.sync_copy(rows_lv, out_lv.at[pl.ds(start, CHUNK)])
        return carry

    lax.fori_loop(0, N_CHUNKS, body, 0)


@jax.jit
def kernel(x, embed_mu, embed_logvar):
    x_flat = x.reshape(B_TOTAL // IDX_W, IDX_W).astype(jnp.int32)

    mesh = plsc.VectorSubcoreMesh(core_axis_name="c", subcore_axis_name="s")
    f = pl.kernel(
        _gather_kernel,
        out_type=(
            jax.ShapeDtypeStruct((B_TOTAL, Z_N), jnp.float32),
            jax.ShapeDtypeStruct((B_TOTAL, Z_N), jnp.float32),
        ),
        mesh=mesh,
        scratch_types=[
            pltpu.VMEM((SUB, IDX_W), jnp.int32),
            pltpu.VMEM((CHUNK, Z_N), jnp.float32),
            pltpu.VMEM((CHUNK, Z_N), jnp.float32),
            pltpu.SemaphoreType.DMA,
        ],
        compiler_params=pltpu.CompilerParams(use_tc_tiling_on_sc=False),
    )
    mu_flat, lv_flat = f(x_flat, embed_mu, embed_logvar)
    mu = mu_flat.reshape(BATCH, HIST, Z_N)
    logvar = lv_flat.reshape(BATCH, HIST, Z_N)
    return (mu, mu, logvar)


# speedup vs baseline: 2.6766x; 1.0363x over previous
import jax
import jax.numpy as jnp
from jax import lax
from jax.experimental import pallas as pl
from jax.experimental.pallas import tpu as pltpu
from jax.experimental.pallas import tpu_sc as plsc

Z_N = 16
BATCH = 16384
HIST = 200
B_TOTAL = BATCH * HIST

NUM_CORES = 2
NUM_SUBCORES = 16
NUM_WORKERS = NUM_CORES * NUM_SUBCORES

IDX_W = 128
CHUNK = 1280
SUB = CHUNK // IDX_W
B_PER_W = B_TOTAL // NUM_WORKERS
N_CHUNKS = B_PER_W // CHUNK
PAIRS = N_CHUNKS // 2


def _gather_kernel(x_hbm, mu_hbm, lv_hbm, out_mu, out_lv,
                   idx0, idx1, rmu0, rlv0, rmu1, rlv1,
                   sg0, sg1, so0, so1):
    wid = lax.axis_index("s") * NUM_CORES + lax.axis_index("c")
    base = wid * B_PER_W
    row_base = wid * (B_PER_W // IDX_W)
    last = N_CHUNKS - 1

    slots = ((idx0, rmu0, rlv0, sg0, so0),
             (idx1, rmu1, rlv1, sg1, so1))

    def fire_gathers(c, slot):
        idx, rmu, rlv, sg, _ = slots[slot]
        pltpu.sync_copy(x_hbm.at[pl.ds(row_base + c * SUB, SUB)], idx)
        for j in range(SUB):
            dst = pl.ds(j * IDX_W, IDX_W)
            pltpu.async_copy(mu_hbm.at[idx.at[j]], rmu.at[dst], sg)
            pltpu.async_copy(lv_hbm.at[idx.at[j]], rlv.at[dst], sg)

    def wait_gathers(slot):
        idx, rmu, rlv, sg, _ = slots[slot]
        pltpu.make_async_copy(mu_hbm.at[pl.ds(0, CHUNK)], rmu, sg).wait()
        pltpu.make_async_copy(lv_hbm.at[pl.ds(0, CHUNK)], rlv, sg).wait()

    def fire_writes(c, slot):
        _, rmu, rlv, _, so = slots[slot]
        start = base + c * CHUNK
        pltpu.async_copy(rmu, out_mu.at[pl.ds(start, CHUNK)], so)
        pltpu.async_copy(rlv, out_lv.at[pl.ds(start, CHUNK)], so)

    def wait_writes(slot):
        _, rmu, rlv, _, so = slots[slot]
        pltpu.make_async_copy(rmu, out_mu.at[pl.ds(0, CHUNK)], so).wait()
        pltpu.make_async_copy(rlv, out_lv.at[pl.ds(0, CHUNK)], so).wait()

    fire_gathers(0, 0)
    fire_gathers(1, 1)

    def body(g, carry):
        c0 = 2 * g
        c1 = c0 + 1
        wait_gathers(0)
        fire_writes(c0, 0)
        wait_gathers(1)
        fire_writes(c1, 1)
        n0 = jnp.minimum(c0 + 2, last)
        n1 = jnp.minimum(c1 + 2, last)
        wait_writes(0)
        fire_gathers(n0, 0)
        wait_writes(1)
        fire_gathers(n1, 1)
        return carry

    lax.fori_loop(0, PAIRS, body, 0)
    wait_gathers(0)
    wait_gathers(1)


@jax.jit
def kernel(x, embed_mu, embed_logvar):
    x_flat = x.reshape(B_TOTAL // IDX_W, IDX_W).astype(jnp.int32)

    mesh = plsc.VectorSubcoreMesh(core_axis_name="c", subcore_axis_name="s")
    f = pl.kernel(
        _gather_kernel,
        out_type=(
            jax.ShapeDtypeStruct((B_TOTAL, Z_N), jnp.float32),
            jax.ShapeDtypeStruct((B_TOTAL, Z_N), jnp.float32),
        ),
        mesh=mesh,
        scratch_types=[
            pltpu.VMEM((SUB, IDX_W), jnp.int32),
            pltpu.VMEM((SUB, IDX_W), jnp.int32),
            pltpu.VMEM((CHUNK, Z_N), jnp.float32),
            pltpu.VMEM((CHUNK, Z_N), jnp.float32),
            pltpu.VMEM((CHUNK, Z_N), jnp.float32),
            pltpu.VMEM((CHUNK, Z_N), jnp.float32),
            pltpu.SemaphoreType.DMA,
            pltpu.SemaphoreType.DMA,
            pltpu.SemaphoreType.DMA,
            pltpu.SemaphoreType.DMA,
        ],
        compiler_params=pltpu.CompilerParams(use_tc_tiling_on_sc=False),
    )
    mu_flat, lv_flat = f(x_flat, embed_mu, embed_logvar)
    mu = mu_flat.reshape(BATCH, HIST, Z_N)
    logvar = lv_flat.reshape(BATCH, HIST, Z_N)
    return (mu, mu, logvar)
